# baseline (speedup 1.0000x reference)
import functools

import jax
import jax.numpy as jnp
from jax import lax
from jax.experimental import pallas as pl
from jax.experimental.pallas import tpu as pltpu
from jax.experimental.pallas import tpu_sc as plsc

_NUM_EMBEDDINGS = 49
_EMBED = 64
_BATCH = 16384
_HIST = 200
_B = _BATCH * _HIST
_NPAIR = _NUM_EMBEDDINGS * _NUM_EMBEDDINGS
_HPAIRS = _HIST // 2

_NC = 2
_NS = 16
_NW = _NC * _NS
_ROWS_PER_W = _BATCH // _NW
_RCHUNK = 4
_CHUNK = _RCHUNK * _HIST
_HALF = _CHUNK // 2
_N_CHUNKS = _ROWS_PER_W // _RCHUNK

_mesh = plsc.VectorSubcoreMesh(core_axis_name="c", subcore_axis_name="s")


@functools.partial(
    pl.kernel,
    mesh=_mesh,
    out_type=jax.ShapeDtypeStruct((_B // 2, 2 * _EMBED), jnp.float32),
    scratch_types=[
        pltpu.VMEM((_RCHUNK, _HIST), jnp.int32),
        pltpu.VMEM((_RCHUNK, _HIST), jnp.int32),
        pltpu.VMEM((_HALF,), jnp.int32),
        pltpu.VMEM((_HALF,), jnp.int32),
        pltpu.VMEM((_HALF, 2 * _EMBED), jnp.float32),
        pltpu.VMEM((_HALF, 2 * _EMBED), jnp.float32),
        pltpu.VMEM_SHARED((_NPAIR, 2 * _EMBED), jnp.float32),
        pltpu.SemaphoreType.DMA,
        pltpu.SemaphoreType.DMA,
        pltpu.SemaphoreType.DMA,
        pltpu.SemaphoreType.DMA,
        pltpu.SemaphoreType.DMA,
        pltpu.SemaphoreType.DMA,
    ],
    compiler_params=pltpu.CompilerParams(
        use_tc_tiling_on_sc=False, needs_layout_passes=False
    ),
)
def _lookup(idx_hbm, table2_hbm, out_hbm, idx0, idx1, pidx0, pidx1,
            rows0, rows1, table_v, si0, si1, sg0, sg1, ss0, ss1):
    sid = lax.axis_index("s")
    wid = sid * _NC + lax.axis_index("c")
    row_base = wid * _ROWS_PER_W
    pair_base = row_base * _HPAIRS

    idx_v = (idx0, idx1)
    pidx_v = (pidx0, pidx1)
    rows_v = (rows0, rows1)
    sem_i = (si0, si1)
    sem_g = (sg0, sg1)
    sem_s = (ss0, ss1)

    @pl.when(sid == 0)
    def _stage_table():
        pltpu.sync_copy(table2_hbm, table_v)

    plsc.subcore_barrier()

    lanes = lax.iota(jnp.int32, 16)

    def chunk_row(j):
        cj = jnp.minimum(j, _N_CHUNKS - 1)
        return row_base + cj * _RCHUNK

    def start_idx(j, b):
        pltpu.async_copy(idx_hbm.at[pl.ds(chunk_row(j), _RCHUNK)], idx_v[b], sem_i[b])

    def wait_idx(b):
        pltpu.make_async_copy(
            idx_hbm.at[pl.ds(row_base, _RCHUNK)], idx_v[b], sem_i[b]
        ).wait()

    def compute_pairs(b):
        for m in range(_HALF // 16):
            k = lanes + (16 * m)
            r = k // _HPAIRS
            c = (k - r * _HPAIRS) * 2
            ev = plsc.load_gather(idx_v[b], [r, c])
            od = plsc.load_gather(idx_v[b], [r, c + 1])
            pidx_v[b][pl.ds(16 * m, 16)] = ev * _NUM_EMBEDDINGS + od

    def start_gather(b):
        pltpu.async_copy(table_v.at[pidx_v[b]], rows_v[b], sem_g[b])

    def wait_gather(b):
        pltpu.make_async_copy(table_v.at[pidx_v[b]], rows_v[b], sem_g[b]).wait()

    def start_scatter(j, b):
        dst = out_hbm.at[pl.ds(pair_base + j * _HALF, _HALF)]
        pltpu.async_copy(rows_v[b], dst, sem_s[b])

    def wait_scatter(b):
        dst = out_hbm.at[pl.ds(pair_base, _HALF)]
        pltpu.make_async_copy(rows_v[b], dst, sem_s[b]).wait()

    start_idx(0, 0)
    start_idx(1, 1)
    wait_idx(0)
    compute_pairs(0)
    start_gather(0)

    wait_gather(0)
    start_scatter(0, 0)
    start_idx(2, 0)
    wait_idx(1)
    compute_pairs(1)
    start_gather(1)

    wait_gather(1)
    start_scatter(1, 1)
    start_idx(3, 1)
    wait_scatter(0)
    wait_idx(0)
    compute_pairs(0)
    start_gather(0)

    def body(g, carry):
        for b in (0, 1):
            j = 2 * g + b
            b1 = 1 - b
            wait_gather(b)
            start_scatter(j, b)
            start_idx(j + 2, b)
            wait_scatter(b1)
            wait_idx(b1)
            compute_pairs(b1)
            start_gather(b1)
        return carry

    lax.fori_loop(1, _N_CHUNKS // 2, body, 0)

    wait_gather(0)
    wait_scatter(1)
    wait_idx(1)


_BB = 64


def _relayout_body(x_ref, o_ref):
    x = x_ref[...]
    a = x[:, :_EMBED]
    b = x[:, _EMBED:]
    o_ref[...] = jnp.stack([a, b], axis=1).reshape(_BB, _HIST, _EMBED)


_relayout = pl.pallas_call(
    _relayout_body,
    grid=(_BATCH // _BB,),
    in_specs=[
        pl.BlockSpec((_BB * _HPAIRS, 2 * _EMBED), lambda i: (i, 0)),
    ],
    out_specs=pl.BlockSpec((_BB, _HIST, _EMBED), lambda i: (i, 0, 0)),
    out_shape=jax.ShapeDtypeStruct((_BATCH, _HIST, _EMBED), jnp.float32),
)


def kernel(time, table):
    table2 = jnp.concatenate(
        [
            jnp.broadcast_to(table[:, None, :], (_NUM_EMBEDDINGS, _NUM_EMBEDDINGS, _EMBED)),
            jnp.broadcast_to(table[None, :, :], (_NUM_EMBEDDINGS, _NUM_EMBEDDINGS, _EMBED)),
        ],
        axis=-1,
    ).reshape(_NPAIR, 2 * _EMBED)
    flat = _lookup(time, table2)
    return _relayout(flat)


# speedup vs baseline: 1.4134x; 1.4134x over previous
import functools

import jax
import jax.numpy as jnp
from jax import lax
from jax.experimental import pallas as pl
from jax.experimental.pallas import tpu as pltpu
from jax.experimental.pallas import tpu_sc as plsc

_NUM_EMBEDDINGS = 49
_EMBED = 64
_BATCH = 16384
_HIST = 200
_B = _BATCH * _HIST
_NPAIR = _NUM_EMBEDDINGS * _NUM_EMBEDDINGS
_HPAIRS = _HIST // 2

_NC = 2
_NS = 16
_NW = _NC * _NS
_ROWS_PER_W = _BATCH // _NW
_RCHUNK = 4
_CHUNK = _RCHUNK * _HIST
_HALF = _CHUNK // 2
_N_CHUNKS = _ROWS_PER_W // _RCHUNK

_mesh = plsc.VectorSubcoreMesh(core_axis_name="c", subcore_axis_name="s")


@functools.partial(
    pl.kernel,
    mesh=_mesh,
    out_type=jax.ShapeDtypeStruct((_B // 2, 2 * _EMBED), jnp.float32),
    scratch_types=[
        pltpu.VMEM((_RCHUNK, _HIST), jnp.int32),
        pltpu.VMEM((_RCHUNK, _HIST), jnp.int32),
        pltpu.VMEM((_HALF,), jnp.int32),
        pltpu.VMEM((_HALF,), jnp.int32),
        pltpu.VMEM((_HALF, 2 * _EMBED), jnp.float32),
        pltpu.VMEM((_HALF, 2 * _EMBED), jnp.float32),
        pltpu.VMEM_SHARED((_NPAIR, 2 * _EMBED), jnp.float32),
        pltpu.SemaphoreType.DMA,
        pltpu.SemaphoreType.DMA,
        pltpu.SemaphoreType.DMA,
        pltpu.SemaphoreType.DMA,
        pltpu.SemaphoreType.DMA,
        pltpu.SemaphoreType.DMA,
    ],
    compiler_params=pltpu.CompilerParams(
        use_tc_tiling_on_sc=True, needs_layout_passes=False
    ),
)
def _lookup(idx_hbm, table2_hbm, out_hbm, idx0, idx1, pidx0, pidx1,
            rows0, rows1, table_v, si0, si1, sg0, sg1, ss0, ss1):
    sid = lax.axis_index("s")
    wid = sid * _NC + lax.axis_index("c")
    row_base = wid * _ROWS_PER_W
    pair_base = row_base * _HPAIRS

    idx_v = (idx0, idx1)
    pidx_v = (pidx0, pidx1)
    rows_v = (rows0, rows1)
    sem_i = (si0, si1)
    sem_g = (sg0, sg1)
    sem_s = (ss0, ss1)

    @pl.when(sid == 0)
    def _stage_table():
        pltpu.sync_copy(table2_hbm, table_v)

    plsc.subcore_barrier()

    lanes = lax.iota(jnp.int32, 16)

    def chunk_row(j):
        cj = jnp.minimum(j, _N_CHUNKS - 1)
        return row_base + cj * _RCHUNK

    def start_idx(j, b):
        pltpu.async_copy(idx_hbm.at[pl.ds(chunk_row(j), _RCHUNK)], idx_v[b], sem_i[b])

    def wait_idx(b):
        pltpu.make_async_copy(
            idx_hbm.at[pl.ds(row_base, _RCHUNK)], idx_v[b], sem_i[b]
        ).wait()

    def compute_pairs(b):
        for m in range(_HALF // 16):
            k = lanes + (16 * m)
            r = k // _HPAIRS
            c = (k - r * _HPAIRS) * 2
            ev = plsc.load_gather(idx_v[b], [r, c])
            od = plsc.load_gather(idx_v[b], [r, c + 1])
            pidx_v[b][pl.ds(16 * m, 16)] = ev * _NUM_EMBEDDINGS + od

    def start_gather(b):
        pltpu.async_copy(table_v.at[pidx_v[b]], rows_v[b], sem_g[b])

    def wait_gather(b):
        pltpu.make_async_copy(table_v.at[pidx_v[b]], rows_v[b], sem_g[b]).wait()

    def start_scatter(j, b):
        dst = out_hbm.at[pl.ds(pair_base + j * _HALF, _HALF)]
        pltpu.async_copy(rows_v[b], dst, sem_s[b])

    def wait_scatter(b):
        dst = out_hbm.at[pl.ds(pair_base, _HALF)]
        pltpu.make_async_copy(rows_v[b], dst, sem_s[b]).wait()

    start_idx(0, 0)
    start_idx(1, 1)
    wait_idx(0)
    compute_pairs(0)
    start_gather(0)

    wait_gather(0)
    start_scatter(0, 0)
    start_idx(2, 0)
    wait_idx(1)
    compute_pairs(1)
    start_gather(1)

    wait_gather(1)
    start_scatter(1, 1)
    start_idx(3, 1)
    wait_scatter(0)
    wait_idx(0)
    compute_pairs(0)
    start_gather(0)

    def body(g, carry):
        for b in (0, 1):
            j = 2 * g + b
            b1 = 1 - b
            wait_gather(b)
            start_scatter(j, b)
            start_idx(j + 2, b)
            wait_scatter(b1)
            wait_idx(b1)
            compute_pairs(b1)
            start_gather(b1)
        return carry

    lax.fori_loop(1, _N_CHUNKS // 2, body, 0)

    wait_gather(0)
    wait_scatter(1)
    wait_idx(1)


def kernel(time, table):
    table2 = jnp.concatenate(
        [
            jnp.broadcast_to(table[:, None, :], (_NUM_EMBEDDINGS, _NUM_EMBEDDINGS, _EMBED)),
            jnp.broadcast_to(table[None, :, :], (_NUM_EMBEDDINGS, _NUM_EMBEDDINGS, _EMBED)),
        ],
        axis=-1,
    ).reshape(_NPAIR, 2 * _EMBED)
    flat = _lookup(time, table2)
    return flat.reshape(_BATCH, _HIST, _EMBED)
